# baseline (speedup 1.0000x reference)
import jax
import jax.numpy as jnp
from jax import lax
from jax.experimental import pallas as pl
from jax.experimental.pallas import tpu as pltpu
from jax.experimental.pallas import tpu_sc as plsc

N = 100_000
E = 3_200_000
NC, NS = 2, 16
NW = NC * NS
ROWS = 784
EPT = ROWS * 128
EPAD = NW * EPT
T = 100_352
CH = 8
NCH = ROWS // CH
TSL = T // NS
GR = T // 128

_SC_ROW_PARAMS = pltpu.CompilerParams(use_tc_tiling_on_sc=False)


def _mesh():
    return plsc.VectorSubcoreMesh(core_axis_name="c", subcore_axis_name="s",
                                  num_cores=NC, num_subcores=NS)


def _edge_pipeline(wid, src_hbm, dst_hbm, sidx_v, didx_v, streams,
                   semi, semg, sems):
    gather = any(t is not None for t, _, _ in streams)

    def fire_idx(c, b):
        if gather:
            pltpu.async_copy(src_hbm.at[wid, pl.ds(c * CH, CH)],
                             sidx_v.at[b], semi)
        pltpu.async_copy(dst_hbm.at[wid, pl.ds(c * CH, CH)],
                         didx_v.at[b], semi)

    def wait_idx(c, b):
        if gather:
            pltpu.make_async_copy(src_hbm.at[wid, pl.ds(c * CH, CH)],
                                  sidx_v.at[b], semi).wait()
        pltpu.make_async_copy(dst_hbm.at[wid, pl.ds(c * CH, CH)],
                              didx_v.at[b], semi).wait()

    def val_ref(tab_sp, vals_v, b, j):
        return vals_v.at[b, j] if tab_sp is not None else vals_v.at[0]

    def drain_scatters(b):
        for tab_sp, vals_v, acc_sp in streams:
            for j in range(CH):
                pltpu.make_async_copy(val_ref(tab_sp, vals_v, b, j),
                                      acc_sp.at[didx_v.at[b, j]],
                                      sems).wait()

    fire_idx(0, 0)

    @pl.loop(0, NCH)
    def _(c):
        b = lax.rem(c, 2)
        wait_idx(c, b)
        for tab_sp, vals_v, _ in streams:
            if tab_sp is not None:
                for j in range(CH):
                    pltpu.async_copy(tab_sp.at[sidx_v.at[b, j]],
                                     vals_v.at[b, j], semg)

        @pl.when(c > 0)
        def _():
            drain_scatters(1 - b)

        @pl.when(c < NCH - 1)
        def _():
            fire_idx(c + 1, 1 - b)

        for tab_sp, vals_v, _ in streams:
            if tab_sp is not None:
                for j in range(CH):
                    pltpu.make_async_copy(tab_sp.at[sidx_v.at[b, j]],
                                          vals_v.at[b, j], semg).wait()
        for tab_sp, vals_v, acc_sp in streams:
            for j in range(CH):
                pltpu.async_copy(val_ref(tab_sp, vals_v, b, j),
                                 acc_sp.at[didx_v.at[b, j]], sems, add=True)

    drain_scatters((NCH - 1) % 2)


_IDX2 = pltpu.VMEM((2, CH, 128), jnp.int32)
_VAL2 = pltpu.VMEM((2, CH, 128), jnp.float32)
_SEMS = [pltpu.SemaphoreType.DMA] * 3


def _sc_deg_body(dst_hbm, zeros_hbm, out_hbm,
                 didx_v, ones_v, deg_sp, semi, semg, sems):
    cid = lax.axis_index("c")
    sid = lax.axis_index("s")
    wid = cid * NS + sid
    sl = pl.ds(sid * TSL, TSL)

    for c0 in range(0, 128, 16):
        ones_v[0, pl.ds(c0, 16)] = jnp.full((16,), 1.0, jnp.float32)
    pltpu.sync_copy(zeros_hbm.at[sl], deg_sp.at[sl])
    plsc.subcore_barrier()

    _edge_pipeline(wid, None, dst_hbm, None, didx_v,
                   [(None, ones_v, deg_sp)], semi, semg, sems)

    plsc.subcore_barrier()
    pltpu.sync_copy(deg_sp.at[sl], out_hbm.at[cid, sl])


def _sc_deg(dstp, zeros_t):
    return pl.kernel(
        _sc_deg_body,
        out_type=jax.ShapeDtypeStruct((NC, T), jnp.float32),
        mesh=_mesh(),
        scratch_types=[_IDX2, pltpu.VMEM((1, 128), jnp.float32),
                       pltpu.VMEM_SHARED((T,), jnp.float32)] + _SEMS,
    )(dstp, zeros_t)


def _sc_agg1_body(src_hbm, dst_hbm, s_hbm, zeros_hbm, out_hbm,
                  sidx_v, didx_v, vals_v, s_sp, acc_sp, semi, semg, sems):
    cid = lax.axis_index("c")
    sid = lax.axis_index("s")
    wid = cid * NS + sid
    sl = pl.ds(sid * TSL, TSL)

    pltpu.sync_copy(s_hbm.at[sl], s_sp.at[sl])
    pltpu.sync_copy(zeros_hbm.at[sl], acc_sp.at[sl])
    plsc.subcore_barrier()

    _edge_pipeline(wid, src_hbm, dst_hbm, sidx_v, didx_v,
                   [(s_sp, vals_v, acc_sp)], semi, semg, sems)

    plsc.subcore_barrier()
    pltpu.sync_copy(acc_sp.at[sl], out_hbm.at[cid, sl])


def _sc_agg1(srcp, dstp, s_t, zeros_t):
    return pl.kernel(
        _sc_agg1_body,
        out_type=jax.ShapeDtypeStruct((NC, T), jnp.float32),
        mesh=_mesh(),
        scratch_types=[_IDX2, _IDX2, _VAL2,
                       pltpu.VMEM_SHARED((T,), jnp.float32),
                       pltpu.VMEM_SHARED((T,), jnp.float32)] + _SEMS,
    )(srcp, dstp, s_t, zeros_t)


def _sc_agg2_body(src_hbm, dst_hbm, p_hbm, zeros_hbm, out_hbm,
                  sidx_v, didx_v, vals_v, p_sp, acc_sp, semi, semg, sems):
    cid = lax.axis_index("c")
    sid = lax.axis_index("s")
    wid = cid * NS + sid
    sl = pl.ds(sid * TSL, TSL)

    pltpu.sync_copy(p_hbm.at[sl, :], p_sp.at[sl, :])
    pltpu.sync_copy(zeros_hbm.at[sl, :], acc_sp.at[sl, :])
    plsc.subcore_barrier()

    _edge_pipeline(wid, src_hbm, dst_hbm, sidx_v, didx_v,
                   [(p_sp, vals_v, acc_sp)], semi, semg, sems)

    plsc.subcore_barrier()
    pltpu.sync_copy(acc_sp.at[sl, :], out_hbm.at[cid, sl, :])


def _sc_agg2(srcp, dstp, p_t, zeros2_t):
    return pl.kernel(
        _sc_agg2_body,
        out_type=jax.ShapeDtypeStruct((NC, T, 2), jnp.float32),
        mesh=_mesh(),
        compiler_params=_SC_ROW_PARAMS,
        scratch_types=[_IDX2, _IDX2,
                       pltpu.VMEM((2, CH, 128, 2), jnp.float32),
                       pltpu.VMEM_SHARED((T, 2), jnp.float32),
                       pltpu.VMEM_SHARED((T, 2), jnp.float32)] + _SEMS,
    )(srcp, dstp, p_t, zeros2_t)


def _glue1_body(degp_ref, x_ref, dis_ref, s_ref):
    deg = degp_ref[0] + degp_ref[1] + 1.0
    dis = lax.rsqrt(deg)
    dis_ref[...] = dis
    s_ref[...] = dis * x_ref[...]


def _glue1(degp, xpad):
    return pl.pallas_call(
        _glue1_body,
        out_shape=[jax.ShapeDtypeStruct((GR, 128), jnp.float32)] * 2,
    )(degp.reshape(NC, GR, 128), xpad)


def _glue2_body(accp_ref, s_ref, dis_ref, prm_ref, p0_ref, p1_ref):
    dis = dis_ref[...]
    u = dis * (accp_ref[0] + accp_ref[1] + s_ref[...])
    h0 = u * prm_ref[0] + prm_ref[2]
    h1 = u * prm_ref[1] + prm_ref[3]
    p0_ref[...] = dis * jnp.maximum(h0, 0.0)
    p1_ref[...] = dis * jnp.maximum(h1, 0.0)


def _glue2(accp, s, dis, prm1):
    return pl.pallas_call(
        _glue2_body,
        in_specs=[
            pl.BlockSpec(memory_space=pltpu.MemorySpace.VMEM),
            pl.BlockSpec(memory_space=pltpu.MemorySpace.VMEM),
            pl.BlockSpec(memory_space=pltpu.MemorySpace.VMEM),
            pl.BlockSpec(memory_space=pltpu.MemorySpace.SMEM),
        ],
        out_shape=[jax.ShapeDtypeStruct((GR, 128), jnp.float32)] * 2,
    )(accp.reshape(NC, GR, 128), s, dis, prm1)


def _glue3_body(a0_ref, a1_ref, p0_ref, p1_ref, dis_ref, prm_ref,
                o0_ref, o1_ref):
    dis = dis_ref[...]
    t0 = dis * (a0_ref[0] + a0_ref[1] + p0_ref[...])
    t1 = dis * (a1_ref[0] + a1_ref[1] + p1_ref[...])
    o0 = t0 * prm_ref[0] + t1 * prm_ref[2] + prm_ref[4]
    o1 = t0 * prm_ref[1] + t1 * prm_ref[3] + prm_ref[5]
    m = jnp.maximum(o0, o1)
    lse = m + jnp.log(jnp.exp(o0 - m) + jnp.exp(o1 - m))
    o0_ref[...] = o0 - lse
    o1_ref[...] = o1 - lse


def _glue3(a0, a1, p0, p1, dis, prm2):
    return pl.pallas_call(
        _glue3_body,
        in_specs=[
            pl.BlockSpec(memory_space=pltpu.MemorySpace.VMEM),
            pl.BlockSpec(memory_space=pltpu.MemorySpace.VMEM),
            pl.BlockSpec(memory_space=pltpu.MemorySpace.VMEM),
            pl.BlockSpec(memory_space=pltpu.MemorySpace.VMEM),
            pl.BlockSpec(memory_space=pltpu.MemorySpace.VMEM),
            pl.BlockSpec(memory_space=pltpu.MemorySpace.SMEM),
        ],
        out_shape=[jax.ShapeDtypeStruct((GR, 128), jnp.float32)] * 2,
    )(a0, a1, p0, p1, dis, prm2)


@jax.jit
def kernel(x, edge_index, W1, b1, W2, b2):
    src = edge_index[0].astype(jnp.int32)
    dst = edge_index[1].astype(jnp.int32)
    pad = EPAD - E
    srcp = jnp.pad(src, (0, pad), constant_values=N).reshape(NW, ROWS, 128)
    dstp = jnp.pad(dst, (0, pad), constant_values=N).reshape(NW, ROWS, 128)

    zeros_t = jnp.zeros((T,), jnp.float32)
    zeros2_t = jnp.zeros((T, 2), jnp.float32)
    xpad = jnp.pad(x[:, 0], (0, T - N)).reshape(GR, 128)

    degp = _sc_deg(dstp, zeros_t)
    dis, s = _glue1(degp, xpad)

    acc1p = _sc_agg1(srcp, dstp, s.reshape(T), zeros_t)

    prm1 = jnp.concatenate([W1[0], b1]).astype(jnp.float32)
    p0, p1 = _glue2(acc1p, s, dis, prm1)

    p_t = jnp.stack([p0.reshape(T), p1.reshape(T)], axis=-1)
    acc2p = _sc_agg2(srcp, dstp, p_t, zeros2_t)

    a0 = acc2p[:, :, 0].reshape(NC, GR, 128)
    a1 = acc2p[:, :, 1].reshape(NC, GR, 128)
    prm2 = jnp.concatenate([W2[0], W2[1], b2]).astype(jnp.float32)
    o0, o1 = _glue3(a0, a1, p0, p1, dis, prm2)

    out = jnp.stack([o0.reshape(T)[:N], o1.reshape(T)[:N]], axis=-1)
    return out


# speedup vs baseline: 1.4358x; 1.4358x over previous
import jax
import jax.numpy as jnp
from jax import lax
from jax.experimental import pallas as pl
from jax.experimental.pallas import tpu as pltpu
from jax.experimental.pallas import tpu_sc as plsc

N = 100_000
E = 3_200_000
NC, NS = 2, 16
NW = NC * NS
ROWS = 784
EPT = ROWS * 128
EPAD = NW * EPT
T = 100_352
CH = 8
NCH = ROWS // CH
TSL = T // NS
GR = T // 128

def _mesh():
    return plsc.VectorSubcoreMesh(core_axis_name="c", subcore_axis_name="s",
                                  num_cores=NC, num_subcores=NS)


def _edge_pipeline(wid, src_hbm, dst_hbm, sidx_v, didx_v, streams,
                   semi, semg, sems):
    gather = any(t is not None for t, _, _ in streams)

    def fire_idx(c, b):
        if gather:
            pltpu.async_copy(src_hbm.at[wid, pl.ds(c * CH, CH)],
                             sidx_v.at[b], semi)
        pltpu.async_copy(dst_hbm.at[wid, pl.ds(c * CH, CH)],
                         didx_v.at[b], semi)

    def wait_idx(c, b):
        if gather:
            pltpu.make_async_copy(src_hbm.at[wid, pl.ds(c * CH, CH)],
                                  sidx_v.at[b], semi).wait()
        pltpu.make_async_copy(dst_hbm.at[wid, pl.ds(c * CH, CH)],
                              didx_v.at[b], semi).wait()

    def val_ref(tab_sp, vals_v, b, j):
        return vals_v.at[b, j] if tab_sp is not None else vals_v.at[0]

    def drain_scatters(b):
        for tab_sp, vals_v, acc_sp in streams:
            for j in range(CH):
                pltpu.make_async_copy(val_ref(tab_sp, vals_v, b, j),
                                      acc_sp.at[didx_v.at[b, j]],
                                      sems).wait()

    fire_idx(0, 0)

    @pl.loop(0, NCH)
    def _(c):
        b = lax.rem(c, 2)
        wait_idx(c, b)
        for tab_sp, vals_v, _ in streams:
            if tab_sp is not None:
                for j in range(CH):
                    pltpu.async_copy(tab_sp.at[sidx_v.at[b, j]],
                                     vals_v.at[b, j], semg)

        @pl.when(c > 0)
        def _():
            drain_scatters(1 - b)

        @pl.when(c < NCH - 1)
        def _():
            fire_idx(c + 1, 1 - b)

        for tab_sp, vals_v, _ in streams:
            if tab_sp is not None:
                for j in range(CH):
                    pltpu.make_async_copy(tab_sp.at[sidx_v.at[b, j]],
                                          vals_v.at[b, j], semg).wait()
        for tab_sp, vals_v, acc_sp in streams:
            for j in range(CH):
                pltpu.async_copy(val_ref(tab_sp, vals_v, b, j),
                                 acc_sp.at[didx_v.at[b, j]], sems, add=True)

    drain_scatters((NCH - 1) % 2)


_IDX2 = pltpu.VMEM((2, CH, 128), jnp.int32)
_VAL2 = pltpu.VMEM((2, CH, 128), jnp.float32)
_SEMS = [pltpu.SemaphoreType.DMA] * 3


def _sc_deg_body(dst_hbm, zeros_hbm, out_hbm,
                 didx_v, ones_v, deg_sp, semi, semg, sems):
    cid = lax.axis_index("c")
    sid = lax.axis_index("s")
    wid = cid * NS + sid
    sl = pl.ds(sid * TSL, TSL)

    for c0 in range(0, 128, 16):
        ones_v[0, pl.ds(c0, 16)] = jnp.full((16,), 1.0, jnp.float32)
    pltpu.sync_copy(zeros_hbm.at[sl], deg_sp.at[sl])
    plsc.subcore_barrier()

    _edge_pipeline(wid, None, dst_hbm, None, didx_v,
                   [(None, ones_v, deg_sp)], semi, semg, sems)

    plsc.subcore_barrier()
    pltpu.sync_copy(deg_sp.at[sl], out_hbm.at[cid, sl])


def _sc_deg(dstp, zeros_t):
    return pl.kernel(
        _sc_deg_body,
        out_type=jax.ShapeDtypeStruct((NC, T), jnp.float32),
        mesh=_mesh(),
        scratch_types=[_IDX2, pltpu.VMEM((1, 128), jnp.float32),
                       pltpu.VMEM_SHARED((T,), jnp.float32)] + _SEMS,
    )(dstp, zeros_t)


def _sc_agg1_body(src_hbm, dst_hbm, s_hbm, zeros_hbm, out_hbm,
                  sidx_v, didx_v, vals_v, s_sp, acc_sp, semi, semg, sems):
    cid = lax.axis_index("c")
    sid = lax.axis_index("s")
    wid = cid * NS + sid
    sl = pl.ds(sid * TSL, TSL)

    pltpu.sync_copy(s_hbm.at[sl], s_sp.at[sl])
    pltpu.sync_copy(zeros_hbm.at[sl], acc_sp.at[sl])
    plsc.subcore_barrier()

    _edge_pipeline(wid, src_hbm, dst_hbm, sidx_v, didx_v,
                   [(s_sp, vals_v, acc_sp)], semi, semg, sems)

    plsc.subcore_barrier()
    pltpu.sync_copy(acc_sp.at[sl], out_hbm.at[cid, sl])


def _sc_agg1(srcp, dstp, s_t, zeros_t):
    return pl.kernel(
        _sc_agg1_body,
        out_type=jax.ShapeDtypeStruct((NC, T), jnp.float32),
        mesh=_mesh(),
        scratch_types=[_IDX2, _IDX2, _VAL2,
                       pltpu.VMEM_SHARED((T,), jnp.float32),
                       pltpu.VMEM_SHARED((T,), jnp.float32)] + _SEMS,
    )(srcp, dstp, s_t, zeros_t)


def _sc_agg2_body(src_hbm, dst_hbm, p0_hbm, p1_hbm, zeros_hbm, out_hbm,
                  sidx_v, didx_v, v0_v, v1_v, p0_sp, p1_sp, a0_sp, a1_sp,
                  semi, semg, sems):
    cid = lax.axis_index("c")
    sid = lax.axis_index("s")
    wid = cid * NS + sid
    sl = pl.ds(sid * TSL, TSL)

    pltpu.sync_copy(p0_hbm.at[sl], p0_sp.at[sl])
    pltpu.sync_copy(p1_hbm.at[sl], p1_sp.at[sl])
    pltpu.sync_copy(zeros_hbm.at[sl], a0_sp.at[sl])
    pltpu.sync_copy(zeros_hbm.at[sl], a1_sp.at[sl])
    plsc.subcore_barrier()

    _edge_pipeline(wid, src_hbm, dst_hbm, sidx_v, didx_v,
                   [(p0_sp, v0_v, a0_sp), (p1_sp, v1_v, a1_sp)],
                   semi, semg, sems)

    plsc.subcore_barrier()
    pltpu.sync_copy(a0_sp.at[sl], out_hbm.at[cid, 0, sl])
    pltpu.sync_copy(a1_sp.at[sl], out_hbm.at[cid, 1, sl])


def _sc_agg2(srcp, dstp, p0_t, p1_t, zeros_t):
    return pl.kernel(
        _sc_agg2_body,
        out_type=jax.ShapeDtypeStruct((NC, 2, T), jnp.float32),
        mesh=_mesh(),
        scratch_types=[_IDX2, _IDX2, _VAL2, _VAL2]
        + [pltpu.VMEM_SHARED((T,), jnp.float32)] * 4 + _SEMS,
    )(srcp, dstp, p0_t, p1_t, zeros_t)


def _glue1_body(degp_ref, x_ref, dis_ref, s_ref):
    deg = degp_ref[0] + degp_ref[1] + 1.0
    dis = lax.rsqrt(deg)
    dis_ref[...] = dis
    s_ref[...] = dis * x_ref[...]


def _glue1(degp, xpad):
    return pl.pallas_call(
        _glue1_body,
        out_shape=[jax.ShapeDtypeStruct((GR, 128), jnp.float32)] * 2,
    )(degp.reshape(NC, GR, 128), xpad)


def _glue2_body(accp_ref, s_ref, dis_ref, prm_ref, p0_ref, p1_ref):
    dis = dis_ref[...]
    u = dis * (accp_ref[0] + accp_ref[1] + s_ref[...])
    h0 = u * prm_ref[0] + prm_ref[2]
    h1 = u * prm_ref[1] + prm_ref[3]
    p0_ref[...] = dis * jnp.maximum(h0, 0.0)
    p1_ref[...] = dis * jnp.maximum(h1, 0.0)


def _glue2(accp, s, dis, prm1):
    return pl.pallas_call(
        _glue2_body,
        in_specs=[
            pl.BlockSpec(memory_space=pltpu.MemorySpace.VMEM),
            pl.BlockSpec(memory_space=pltpu.MemorySpace.VMEM),
            pl.BlockSpec(memory_space=pltpu.MemorySpace.VMEM),
            pl.BlockSpec(memory_space=pltpu.MemorySpace.SMEM),
        ],
        out_shape=[jax.ShapeDtypeStruct((GR, 128), jnp.float32)] * 2,
    )(accp.reshape(NC, GR, 128), s, dis, prm1)


def _glue3_body(a0_ref, a1_ref, p0_ref, p1_ref, dis_ref, prm_ref,
                o0_ref, o1_ref):
    dis = dis_ref[...]
    t0 = dis * (a0_ref[0] + a0_ref[1] + p0_ref[...])
    t1 = dis * (a1_ref[0] + a1_ref[1] + p1_ref[...])
    o0 = t0 * prm_ref[0] + t1 * prm_ref[2] + prm_ref[4]
    o1 = t0 * prm_ref[1] + t1 * prm_ref[3] + prm_ref[5]
    m = jnp.maximum(o0, o1)
    lse = m + jnp.log(jnp.exp(o0 - m) + jnp.exp(o1 - m))
    o0_ref[...] = o0 - lse
    o1_ref[...] = o1 - lse


def _glue3(a0, a1, p0, p1, dis, prm2):
    return pl.pallas_call(
        _glue3_body,
        in_specs=[
            pl.BlockSpec(memory_space=pltpu.MemorySpace.VMEM),
            pl.BlockSpec(memory_space=pltpu.MemorySpace.VMEM),
            pl.BlockSpec(memory_space=pltpu.MemorySpace.VMEM),
            pl.BlockSpec(memory_space=pltpu.MemorySpace.VMEM),
            pl.BlockSpec(memory_space=pltpu.MemorySpace.VMEM),
            pl.BlockSpec(memory_space=pltpu.MemorySpace.SMEM),
        ],
        out_shape=[jax.ShapeDtypeStruct((GR, 128), jnp.float32)] * 2,
    )(a0, a1, p0, p1, dis, prm2)


@jax.jit
def kernel(x, edge_index, W1, b1, W2, b2):
    src = edge_index[0].astype(jnp.int32)
    dst = edge_index[1].astype(jnp.int32)
    pad = EPAD - E
    srcp = jnp.pad(src, (0, pad), constant_values=N).reshape(NW, ROWS, 128)
    dstp = jnp.pad(dst, (0, pad), constant_values=N).reshape(NW, ROWS, 128)

    zeros_t = jnp.zeros((T,), jnp.float32)
    xpad = jnp.pad(x[:, 0], (0, T - N)).reshape(GR, 128)

    degp = _sc_deg(dstp, zeros_t)
    dis, s = _glue1(degp, xpad)

    acc1p = _sc_agg1(srcp, dstp, s.reshape(T), zeros_t)

    prm1 = jnp.concatenate([W1[0], b1]).astype(jnp.float32)
    p0, p1 = _glue2(acc1p, s, dis, prm1)

    acc2p = _sc_agg2(srcp, dstp, p0.reshape(T), p1.reshape(T), zeros_t)

    a0 = acc2p[:, 0, :].reshape(NC, GR, 128)
    a1 = acc2p[:, 1, :].reshape(NC, GR, 128)
    prm2 = jnp.concatenate([W2[0], W2[1], b2]).astype(jnp.float32)
    o0, o1 = _glue3(a0, a1, p0, p1, dis, prm2)

    out = jnp.stack([o0.reshape(T)[:N], o1.reshape(T)[:N]], axis=-1)
    return out


# speedup vs baseline: 1.6090x; 1.1206x over previous
import jax
import jax.numpy as jnp
from jax import lax
from jax.experimental import pallas as pl
from jax.experimental.pallas import tpu as pltpu
from jax.experimental.pallas import tpu_sc as plsc

N = 100_000
E = 3_200_000
NC, NS = 2, 16
NW = NC * NS
ROWS = 784
RR = E // 128
T = 100_352
CH = 8
NB = 3
NCH = ROWS // CH
TSL = T // NS
GR = T // 128

def _mesh():
    return plsc.VectorSubcoreMesh(core_axis_name="c", subcore_axis_name="s",
                                  num_cores=NC, num_subcores=NS)


def _edge_pipeline(wid, src_hbm, dst_hbm, dum_hbm, sidx_v, didx_v, streams,
                   semi, semg, sems):
    gather = any(t is not None for t, _, _ in streams)

    def fire_idx(c, b):
        row0 = wid * ROWS + c * CH

        @pl.when(row0 < RR)
        def _():
            if gather:
                pltpu.async_copy(src_hbm.at[pl.ds(row0, CH)],
                                 sidx_v.at[b], semi)
            pltpu.async_copy(dst_hbm.at[pl.ds(row0, CH)],
                             didx_v.at[b], semi)

        @pl.when(row0 >= RR)
        def _():
            if gather:
                pltpu.async_copy(dum_hbm.at[pl.ds(0, CH)],
                                 sidx_v.at[b], semi)
            pltpu.async_copy(dum_hbm.at[pl.ds(0, CH)],
                             didx_v.at[b], semi)

    def wait_idx(c, b):
        row0 = wid * ROWS + c * CH

        @pl.when(row0 < RR)
        def _():
            if gather:
                pltpu.make_async_copy(src_hbm.at[pl.ds(row0, CH)],
                                      sidx_v.at[b], semi).wait()
            pltpu.make_async_copy(dst_hbm.at[pl.ds(row0, CH)],
                                  didx_v.at[b], semi).wait()

        @pl.when(row0 >= RR)
        def _():
            if gather:
                pltpu.make_async_copy(dum_hbm.at[pl.ds(0, CH)],
                                      sidx_v.at[b], semi).wait()
            pltpu.make_async_copy(dum_hbm.at[pl.ds(0, CH)],
                                  didx_v.at[b], semi).wait()

    def val_ref(tab_sp, vals_v, b, j):
        return vals_v.at[b, j] if tab_sp is not None else vals_v.at[0]

    def drain_scatters(b):
        for tab_sp, vals_v, acc_sp in streams:
            for j in range(CH):
                pltpu.make_async_copy(val_ref(tab_sp, vals_v, b, j),
                                      acc_sp.at[didx_v.at[b, j]],
                                      sems).wait()

    fire_idx(0, 0)

    @pl.loop(0, NCH)
    def _(c):
        b = lax.rem(c, NB)
        wait_idx(c, b)
        for tab_sp, vals_v, _ in streams:
            if tab_sp is not None:
                for j in range(CH):
                    pltpu.async_copy(tab_sp.at[sidx_v.at[b, j]],
                                     vals_v.at[b, j], semg)

        @pl.when(c >= NB - 1)
        def _():
            drain_scatters(lax.rem(c + 1, NB))

        @pl.when(c < NCH - 1)
        def _():
            fire_idx(c + 1, lax.rem(c + 1, NB))

        for tab_sp, vals_v, _ in streams:
            if tab_sp is not None:
                for j in range(CH):
                    pltpu.make_async_copy(tab_sp.at[sidx_v.at[b, j]],
                                          vals_v.at[b, j], semg).wait()
        for tab_sp, vals_v, acc_sp in streams:
            for j in range(CH):
                pltpu.async_copy(val_ref(tab_sp, vals_v, b, j),
                                 acc_sp.at[didx_v.at[b, j]], sems, add=True)

    for cc in range(NCH - NB + 1, NCH):
        drain_scatters(cc % NB)


_IDX2 = pltpu.VMEM((NB, CH, 128), jnp.int32)
_VAL2 = pltpu.VMEM((NB, CH, 128), jnp.float32)
_SEMS = [pltpu.SemaphoreType.DMA] * 3


def _sc_deg_body(dst_hbm, dum_hbm, zeros_hbm, out_hbm,
                 didx_v, ones_v, deg_sp, semi, semg, sems):
    cid = lax.axis_index("c")
    sid = lax.axis_index("s")
    wid = cid * NS + sid
    sl = pl.ds(sid * TSL, TSL)

    for c0 in range(0, 128, 16):
        ones_v[0, pl.ds(c0, 16)] = jnp.full((16,), 1.0, jnp.float32)
    pltpu.sync_copy(zeros_hbm.at[sl], deg_sp.at[sl])
    plsc.subcore_barrier()

    _edge_pipeline(wid, None, dst_hbm, dum_hbm, None, didx_v,
                   [(None, ones_v, deg_sp)], semi, semg, sems)

    plsc.subcore_barrier()
    pltpu.sync_copy(deg_sp.at[sl], out_hbm.at[cid, sl])


def _sc_deg(dstp, dum, zeros_t):
    return pl.kernel(
        _sc_deg_body,
        out_type=jax.ShapeDtypeStruct((NC, T), jnp.float32),
        mesh=_mesh(),
        scratch_types=[_IDX2, pltpu.VMEM((1, 128), jnp.float32),
                       pltpu.VMEM_SHARED((T,), jnp.float32)] + _SEMS,
    )(dstp, dum, zeros_t)


def _sc_agg1_body(src_hbm, dst_hbm, dum_hbm, s_hbm, zeros_hbm, out_hbm,
                  sidx_v, didx_v, vals_v, s_sp, acc_sp, semi, semg, sems):
    cid = lax.axis_index("c")
    sid = lax.axis_index("s")
    wid = cid * NS + sid
    sl = pl.ds(sid * TSL, TSL)

    pltpu.sync_copy(s_hbm.at[sl], s_sp.at[sl])
    pltpu.sync_copy(zeros_hbm.at[sl], acc_sp.at[sl])
    plsc.subcore_barrier()

    _edge_pipeline(wid, src_hbm, dst_hbm, dum_hbm, sidx_v, didx_v,
                   [(s_sp, vals_v, acc_sp)], semi, semg, sems)

    plsc.subcore_barrier()
    pltpu.sync_copy(acc_sp.at[sl], out_hbm.at[cid, sl])


def _sc_agg1(srcp, dstp, dum, s_t, zeros_t):
    return pl.kernel(
        _sc_agg1_body,
        out_type=jax.ShapeDtypeStruct((NC, T), jnp.float32),
        mesh=_mesh(),
        scratch_types=[_IDX2, _IDX2, _VAL2,
                       pltpu.VMEM_SHARED((T,), jnp.float32),
                       pltpu.VMEM_SHARED((T,), jnp.float32)] + _SEMS,
    )(srcp, dstp, dum, s_t, zeros_t)


def _sc_agg2_body(src_hbm, dst_hbm, dum_hbm, p0_hbm, p1_hbm, zeros_hbm,
                  out_hbm, sidx_v, didx_v, v0_v, v1_v, p0_sp, p1_sp, a0_sp,
                  a1_sp, semi, semg, sems):
    cid = lax.axis_index("c")
    sid = lax.axis_index("s")
    wid = cid * NS + sid
    sl = pl.ds(sid * TSL, TSL)

    pltpu.sync_copy(p0_hbm.at[sl], p0_sp.at[sl])
    pltpu.sync_copy(p1_hbm.at[sl], p1_sp.at[sl])
    pltpu.sync_copy(zeros_hbm.at[sl], a0_sp.at[sl])
    pltpu.sync_copy(zeros_hbm.at[sl], a1_sp.at[sl])
    plsc.subcore_barrier()

    _edge_pipeline(wid, src_hbm, dst_hbm, dum_hbm, sidx_v, didx_v,
                   [(p0_sp, v0_v, a0_sp), (p1_sp, v1_v, a1_sp)],
                   semi, semg, sems)

    plsc.subcore_barrier()
    pltpu.sync_copy(a0_sp.at[sl], out_hbm.at[cid, 0, sl])
    pltpu.sync_copy(a1_sp.at[sl], out_hbm.at[cid, 1, sl])


def _sc_agg2(srcp, dstp, dum, p0_t, p1_t, zeros_t):
    return pl.kernel(
        _sc_agg2_body,
        out_type=jax.ShapeDtypeStruct((NC, 2, T), jnp.float32),
        mesh=_mesh(),
        scratch_types=[_IDX2, _IDX2, _VAL2, _VAL2]
        + [pltpu.VMEM_SHARED((T,), jnp.float32)] * 4 + _SEMS,
    )(srcp, dstp, dum, p0_t, p1_t, zeros_t)


def _glue1_body(degp_ref, x_ref, dis_ref, s_ref):
    deg = degp_ref[0] + degp_ref[1] + 1.0
    dis = lax.rsqrt(deg)
    dis_ref[...] = dis
    s_ref[...] = dis * x_ref[...]


def _glue1(degp, xpad):
    return pl.pallas_call(
        _glue1_body,
        out_shape=[jax.ShapeDtypeStruct((GR, 128), jnp.float32)] * 2,
    )(degp.reshape(NC, GR, 128), xpad)


def _glue2_body(accp_ref, s_ref, dis_ref, prm_ref, p0_ref, p1_ref):
    dis = dis_ref[...]
    u = dis * (accp_ref[0] + accp_ref[1] + s_ref[...])
    h0 = u * prm_ref[0] + prm_ref[2]
    h1 = u * prm_ref[1] + prm_ref[3]
    p0_ref[...] = dis * jnp.maximum(h0, 0.0)
    p1_ref[...] = dis * jnp.maximum(h1, 0.0)


def _glue2(accp, s, dis, prm1):
    return pl.pallas_call(
        _glue2_body,
        in_specs=[
            pl.BlockSpec(memory_space=pltpu.MemorySpace.VMEM),
            pl.BlockSpec(memory_space=pltpu.MemorySpace.VMEM),
            pl.BlockSpec(memory_space=pltpu.MemorySpace.VMEM),
            pl.BlockSpec(memory_space=pltpu.MemorySpace.SMEM),
        ],
        out_shape=[jax.ShapeDtypeStruct((GR, 128), jnp.float32)] * 2,
    )(accp.reshape(NC, GR, 128), s, dis, prm1)


def _glue3_body(a0_ref, a1_ref, p0_ref, p1_ref, dis_ref, prm_ref,
                o0_ref, o1_ref):
    dis = dis_ref[...]
    t0 = dis * (a0_ref[0] + a0_ref[1] + p0_ref[...])
    t1 = dis * (a1_ref[0] + a1_ref[1] + p1_ref[...])
    o0 = t0 * prm_ref[0] + t1 * prm_ref[2] + prm_ref[4]
    o1 = t0 * prm_ref[1] + t1 * prm_ref[3] + prm_ref[5]
    m = jnp.maximum(o0, o1)
    lse = m + jnp.log(jnp.exp(o0 - m) + jnp.exp(o1 - m))
    o0_ref[...] = o0 - lse
    o1_ref[...] = o1 - lse


def _glue3(a0, a1, p0, p1, dis, prm2):
    return pl.pallas_call(
        _glue3_body,
        in_specs=[
            pl.BlockSpec(memory_space=pltpu.MemorySpace.VMEM),
            pl.BlockSpec(memory_space=pltpu.MemorySpace.VMEM),
            pl.BlockSpec(memory_space=pltpu.MemorySpace.VMEM),
            pl.BlockSpec(memory_space=pltpu.MemorySpace.VMEM),
            pl.BlockSpec(memory_space=pltpu.MemorySpace.VMEM),
            pl.BlockSpec(memory_space=pltpu.MemorySpace.SMEM),
        ],
        out_shape=[jax.ShapeDtypeStruct((GR, 128), jnp.float32)] * 2,
    )(a0, a1, p0, p1, dis, prm2)


@jax.jit
def kernel(x, edge_index, W1, b1, W2, b2):
    srcp = edge_index[0].astype(jnp.int32).reshape(RR, 128)
    dstp = edge_index[1].astype(jnp.int32).reshape(RR, 128)
    dum = jnp.full((CH, 128), N, jnp.int32)

    zeros_t = jnp.zeros((T,), jnp.float32)
    xpad = jnp.pad(x[:, 0], (0, T - N)).reshape(GR, 128)

    degp = _sc_deg(dstp, dum, zeros_t)
    dis, s = _glue1(degp, xpad)

    acc1p = _sc_agg1(srcp, dstp, dum, s.reshape(T), zeros_t)

    prm1 = jnp.concatenate([W1[0], b1]).astype(jnp.float32)
    p0, p1 = _glue2(acc1p, s, dis, prm1)

    acc2p = _sc_agg2(srcp, dstp, dum, p0.reshape(T), p1.reshape(T), zeros_t)

    a0 = acc2p[:, 0, :].reshape(NC, GR, 128)
    a1 = acc2p[:, 1, :].reshape(NC, GR, 128)
    prm2 = jnp.concatenate([W2[0], W2[1], b2]).astype(jnp.float32)
    o0, o1 = _glue3(a0, a1, p0, p1, dis, prm2)

    out = jnp.stack([o0.reshape(T)[:N], o1.reshape(T)[:N]], axis=-1)
    return out


# speedup vs baseline: 1.6218x; 1.0080x over previous
import jax
import jax.numpy as jnp
from jax import lax
from jax.experimental import pallas as pl
from jax.experimental.pallas import tpu as pltpu
from jax.experimental.pallas import tpu_sc as plsc

N = 100_000
E = 3_200_000
NC, NS = 2, 16
NW = NC * NS
ROWS = 784
RR = E // 128
T = 100_352
CH = 8
NB = 4
NCH = ROWS // CH
TSL = T // NS
GR = T // 128

def _mesh():
    return plsc.VectorSubcoreMesh(core_axis_name="c", subcore_axis_name="s",
                                  num_cores=NC, num_subcores=NS)


def _edge_pipeline(wid, src_hbm, dst_hbm, dum_hbm, sidx_v, didx_v, streams,
                   semi, semg, sems):
    gather = any(t is not None for t, _, _ in streams)

    def fire_idx(c, b):
        row0 = wid * ROWS + c * CH

        @pl.when(row0 < RR)
        def _():
            if gather:
                pltpu.async_copy(src_hbm.at[pl.ds(row0, CH)],
                                 sidx_v.at[b], semi)
            pltpu.async_copy(dst_hbm.at[pl.ds(row0, CH)],
                             didx_v.at[b], semi)

        @pl.when(row0 >= RR)
        def _():
            if gather:
                pltpu.async_copy(dum_hbm.at[pl.ds(0, CH)],
                                 sidx_v.at[b], semi)
            pltpu.async_copy(dum_hbm.at[pl.ds(0, CH)],
                             didx_v.at[b], semi)

    def wait_idx(c, b):
        row0 = wid * ROWS + c * CH

        @pl.when(row0 < RR)
        def _():
            if gather:
                pltpu.make_async_copy(src_hbm.at[pl.ds(row0, CH)],
                                      sidx_v.at[b], semi).wait()
            pltpu.make_async_copy(dst_hbm.at[pl.ds(row0, CH)],
                                  didx_v.at[b], semi).wait()

        @pl.when(row0 >= RR)
        def _():
            if gather:
                pltpu.make_async_copy(dum_hbm.at[pl.ds(0, CH)],
                                      sidx_v.at[b], semi).wait()
            pltpu.make_async_copy(dum_hbm.at[pl.ds(0, CH)],
                                  didx_v.at[b], semi).wait()

    def val_ref(tab_sp, vals_v, b, j):
        return vals_v.at[b, j] if tab_sp is not None else vals_v.at[0]

    def drain_scatters(b):
        for tab_sp, vals_v, acc_sp in streams:
            for j in range(CH):
                pltpu.make_async_copy(val_ref(tab_sp, vals_v, b, j),
                                      acc_sp.at[didx_v.at[b, j]],
                                      sems).wait()

    fire_idx(0, 0)

    @pl.loop(0, NCH)
    def _(c):
        b = lax.rem(c, NB)
        wait_idx(c, b)
        for tab_sp, vals_v, _ in streams:
            if tab_sp is not None:
                for j in range(CH):
                    pltpu.async_copy(tab_sp.at[sidx_v.at[b, j]],
                                     vals_v.at[b, j], semg)

        @pl.when(c >= NB - 1)
        def _():
            drain_scatters(lax.rem(c + 1, NB))

        @pl.when(c < NCH - 1)
        def _():
            fire_idx(c + 1, lax.rem(c + 1, NB))

        for tab_sp, vals_v, _ in streams:
            if tab_sp is not None:
                for j in range(CH):
                    pltpu.make_async_copy(tab_sp.at[sidx_v.at[b, j]],
                                          vals_v.at[b, j], semg).wait()
        for tab_sp, vals_v, acc_sp in streams:
            for j in range(CH):
                pltpu.async_copy(val_ref(tab_sp, vals_v, b, j),
                                 acc_sp.at[didx_v.at[b, j]], sems, add=True)

    for cc in range(NCH - NB + 1, NCH):
        drain_scatters(cc % NB)


_IDX2 = pltpu.VMEM((NB, CH, 128), jnp.int32)
_VAL2 = pltpu.VMEM((NB, CH, 128), jnp.float32)
_SEMS = [pltpu.SemaphoreType.DMA] * 3


def _sc_deg_body(dst_hbm, dum_hbm, zeros_hbm, out_hbm,
                 didx_v, ones_v, deg_sp, semi, semg, sems):
    cid = lax.axis_index("c")
    sid = lax.axis_index("s")
    wid = cid * NS + sid
    sl = pl.ds(sid * TSL, TSL)

    for c0 in range(0, 128, 16):
        ones_v[0, pl.ds(c0, 16)] = jnp.full((16,), 1.0, jnp.float32)
    pltpu.sync_copy(zeros_hbm.at[sl], deg_sp.at[sl])
    plsc.subcore_barrier()

    _edge_pipeline(wid, None, dst_hbm, dum_hbm, None, didx_v,
                   [(None, ones_v, deg_sp)], semi, semg, sems)

    plsc.subcore_barrier()
    pltpu.sync_copy(deg_sp.at[sl], out_hbm.at[cid, sl])


def _sc_deg(dstp, dum, zeros_t):
    return pl.kernel(
        _sc_deg_body,
        out_type=jax.ShapeDtypeStruct((NC, T), jnp.float32),
        mesh=_mesh(),
        scratch_types=[_IDX2, pltpu.VMEM((1, 128), jnp.float32),
                       pltpu.VMEM_SHARED((T,), jnp.float32)] + _SEMS,
    )(dstp, dum, zeros_t)


def _sc_agg1_body(src_hbm, dst_hbm, dum_hbm, s_hbm, zeros_hbm, out_hbm,
                  sidx_v, didx_v, vals_v, s_sp, acc_sp, semi, semg, sems):
    cid = lax.axis_index("c")
    sid = lax.axis_index("s")
    wid = cid * NS + sid
    sl = pl.ds(sid * TSL, TSL)

    pltpu.sync_copy(s_hbm.at[sl], s_sp.at[sl])
    pltpu.sync_copy(zeros_hbm.at[sl], acc_sp.at[sl])
    plsc.subcore_barrier()

    _edge_pipeline(wid, src_hbm, dst_hbm, dum_hbm, sidx_v, didx_v,
                   [(s_sp, vals_v, acc_sp)], semi, semg, sems)

    plsc.subcore_barrier()
    pltpu.sync_copy(acc_sp.at[sl], out_hbm.at[cid, sl])


def _sc_agg1(srcp, dstp, dum, s_t, zeros_t):
    return pl.kernel(
        _sc_agg1_body,
        out_type=jax.ShapeDtypeStruct((NC, T), jnp.float32),
        mesh=_mesh(),
        scratch_types=[_IDX2, _IDX2, _VAL2,
                       pltpu.VMEM_SHARED((T,), jnp.float32),
                       pltpu.VMEM_SHARED((T,), jnp.float32)] + _SEMS,
    )(srcp, dstp, dum, s_t, zeros_t)


def _sc_agg2_body(src_hbm, dst_hbm, dum_hbm, p0_hbm, p1_hbm, zeros_hbm,
                  out_hbm, sidx_v, didx_v, v0_v, v1_v, p0_sp, p1_sp, a0_sp,
                  a1_sp, semi, semg, sems):
    cid = lax.axis_index("c")
    sid = lax.axis_index("s")
    wid = cid * NS + sid
    sl = pl.ds(sid * TSL, TSL)

    pltpu.sync_copy(p0_hbm.at[sl], p0_sp.at[sl])
    pltpu.sync_copy(p1_hbm.at[sl], p1_sp.at[sl])
    pltpu.sync_copy(zeros_hbm.at[sl], a0_sp.at[sl])
    pltpu.sync_copy(zeros_hbm.at[sl], a1_sp.at[sl])
    plsc.subcore_barrier()

    _edge_pipeline(wid, src_hbm, dst_hbm, dum_hbm, sidx_v, didx_v,
                   [(p0_sp, v0_v, a0_sp), (p1_sp, v1_v, a1_sp)],
                   semi, semg, sems)

    plsc.subcore_barrier()
    pltpu.sync_copy(a0_sp.at[sl], out_hbm.at[cid, 0, sl])
    pltpu.sync_copy(a1_sp.at[sl], out_hbm.at[cid, 1, sl])


def _sc_agg2(srcp, dstp, dum, p0_t, p1_t, zeros_t):
    return pl.kernel(
        _sc_agg2_body,
        out_type=jax.ShapeDtypeStruct((NC, 2, T), jnp.float32),
        mesh=_mesh(),
        scratch_types=[_IDX2, _IDX2, _VAL2, _VAL2]
        + [pltpu.VMEM_SHARED((T,), jnp.float32)] * 4 + _SEMS,
    )(srcp, dstp, dum, p0_t, p1_t, zeros_t)


def _glue1_body(degp_ref, x_ref, dis_ref, s_ref):
    deg = degp_ref[0] + degp_ref[1] + 1.0
    dis = lax.rsqrt(deg)
    dis_ref[...] = dis
    s_ref[...] = dis * x_ref[...]


def _glue1(degp, xpad):
    return pl.pallas_call(
        _glue1_body,
        out_shape=[jax.ShapeDtypeStruct((GR, 128), jnp.float32)] * 2,
    )(degp.reshape(NC, GR, 128), xpad)


def _glue2_body(accp_ref, s_ref, dis_ref, prm_ref, p0_ref, p1_ref):
    dis = dis_ref[...]
    u = dis * (accp_ref[0] + accp_ref[1] + s_ref[...])
    h0 = u * prm_ref[0] + prm_ref[2]
    h1 = u * prm_ref[1] + prm_ref[3]
    p0_ref[...] = dis * jnp.maximum(h0, 0.0)
    p1_ref[...] = dis * jnp.maximum(h1, 0.0)


def _glue2(accp, s, dis, prm1):
    return pl.pallas_call(
        _glue2_body,
        in_specs=[
            pl.BlockSpec(memory_space=pltpu.MemorySpace.VMEM),
            pl.BlockSpec(memory_space=pltpu.MemorySpace.VMEM),
            pl.BlockSpec(memory_space=pltpu.MemorySpace.VMEM),
            pl.BlockSpec(memory_space=pltpu.MemorySpace.SMEM),
        ],
        out_shape=[jax.ShapeDtypeStruct((GR, 128), jnp.float32)] * 2,
    )(accp.reshape(NC, GR, 128), s, dis, prm1)


def _glue3_body(a0_ref, a1_ref, p0_ref, p1_ref, dis_ref, prm_ref,
                o0_ref, o1_ref):
    dis = dis_ref[...]
    t0 = dis * (a0_ref[0] + a0_ref[1] + p0_ref[...])
    t1 = dis * (a1_ref[0] + a1_ref[1] + p1_ref[...])
    o0 = t0 * prm_ref[0] + t1 * prm_ref[2] + prm_ref[4]
    o1 = t0 * prm_ref[1] + t1 * prm_ref[3] + prm_ref[5]
    m = jnp.maximum(o0, o1)
    lse = m + jnp.log(jnp.exp(o0 - m) + jnp.exp(o1 - m))
    o0_ref[...] = o0 - lse
    o1_ref[...] = o1 - lse


def _glue3(a0, a1, p0, p1, dis, prm2):
    return pl.pallas_call(
        _glue3_body,
        in_specs=[
            pl.BlockSpec(memory_space=pltpu.MemorySpace.VMEM),
            pl.BlockSpec(memory_space=pltpu.MemorySpace.VMEM),
            pl.BlockSpec(memory_space=pltpu.MemorySpace.VMEM),
            pl.BlockSpec(memory_space=pltpu.MemorySpace.VMEM),
            pl.BlockSpec(memory_space=pltpu.MemorySpace.VMEM),
            pl.BlockSpec(memory_space=pltpu.MemorySpace.SMEM),
        ],
        out_shape=[jax.ShapeDtypeStruct((GR, 128), jnp.float32)] * 2,
    )(a0, a1, p0, p1, dis, prm2)


@jax.jit
def kernel(x, edge_index, W1, b1, W2, b2):
    srcp = edge_index[0].astype(jnp.int32).reshape(RR, 128)
    dstp = edge_index[1].astype(jnp.int32).reshape(RR, 128)
    dum = jnp.full((CH, 128), N, jnp.int32)

    zeros_t = jnp.zeros((T,), jnp.float32)
    xpad = jnp.pad(x[:, 0], (0, T - N)).reshape(GR, 128)

    degp = _sc_deg(dstp, dum, zeros_t)
    dis, s = _glue1(degp, xpad)

    acc1p = _sc_agg1(srcp, dstp, dum, s.reshape(T), zeros_t)

    prm1 = jnp.concatenate([W1[0], b1]).astype(jnp.float32)
    p0, p1 = _glue2(acc1p, s, dis, prm1)

    acc2p = _sc_agg2(srcp, dstp, dum, p0.reshape(T), p1.reshape(T), zeros_t)

    a0 = acc2p[:, 0, :].reshape(NC, GR, 128)
    a1 = acc2p[:, 1, :].reshape(NC, GR, 128)
    prm2 = jnp.concatenate([W2[0], W2[1], b2]).astype(jnp.float32)
    o0, o1 = _glue3(a0, a1, p0, p1, dis, prm2)

    out = jnp.stack([o0.reshape(T)[:N], o1.reshape(T)[:N]], axis=-1)
    return out


# speedup vs baseline: 1.6304x; 1.0053x over previous
import jax
import jax.numpy as jnp
from jax import lax
from jax.experimental import pallas as pl
from jax.experimental.pallas import tpu as pltpu
from jax.experimental.pallas import tpu_sc as plsc

N = 100_000
E = 3_200_000
NC, NS = 2, 16
NW = NC * NS
ROWS = 784
RR = E // 128
T = 100_352
CH = 8
NB = 4
NCH = ROWS // CH
TSL = T // NS
GR = T // 128

def _mesh():
    return plsc.VectorSubcoreMesh(core_axis_name="c", subcore_axis_name="s",
                                  num_cores=NC, num_subcores=NS)


def _edge_pipeline(wid, src_hbm, dst_hbm, dum_hbm, sidx_v, didx_v, streams,
                   semi, semg, sems):
    gather = any(t is not None for t, _, _ in streams)

    def fire_idx(c, b):
        row0 = wid * ROWS + c * CH

        @pl.when(row0 < RR)
        def _():
            if gather:
                pltpu.async_copy(src_hbm.at[pl.ds(row0, CH)],
                                 sidx_v.at[b], semi)
            pltpu.async_copy(dst_hbm.at[pl.ds(row0, CH)],
                             didx_v.at[b], semi)

        @pl.when(row0 >= RR)
        def _():
            if gather:
                pltpu.async_copy(dum_hbm.at[pl.ds(0, CH)],
                                 sidx_v.at[b], semi)
            pltpu.async_copy(dum_hbm.at[pl.ds(0, CH)],
                             didx_v.at[b], semi)

    def wait_idx(c, b):
        row0 = wid * ROWS + c * CH

        @pl.when(row0 < RR)
        def _():
            if gather:
                pltpu.make_async_copy(src_hbm.at[pl.ds(row0, CH)],
                                      sidx_v.at[b], semi).wait()
            pltpu.make_async_copy(dst_hbm.at[pl.ds(row0, CH)],
                                  didx_v.at[b], semi).wait()

        @pl.when(row0 >= RR)
        def _():
            if gather:
                pltpu.make_async_copy(dum_hbm.at[pl.ds(0, CH)],
                                      sidx_v.at[b], semi).wait()
            pltpu.make_async_copy(dum_hbm.at[pl.ds(0, CH)],
                                  didx_v.at[b], semi).wait()

    def val_ref(tab_sp, vals_v, b, j):
        return vals_v.at[b, j] if tab_sp is not None else vals_v.at[0]

    def drain_scatters(b):
        for tab_sp, vals_v, acc_sp in streams:
            for j in range(CH):
                pltpu.make_async_copy(val_ref(tab_sp, vals_v, b, j),
                                      acc_sp.at[didx_v.at[b, j]],
                                      sems).wait()

    fire_idx(0, 0)

    @pl.loop(0, NCH)
    def _(c):
        b = lax.rem(c, NB)
        wait_idx(c, b)
        for tab_sp, vals_v, _ in streams:
            if tab_sp is not None:
                for j in range(CH):
                    pltpu.async_copy(tab_sp.at[sidx_v.at[b, j]],
                                     vals_v.at[b, j], semg)

        @pl.when(c >= NB - 1)
        def _():
            drain_scatters(lax.rem(c + 1, NB))

        @pl.when(c < NCH - 1)
        def _():
            fire_idx(c + 1, lax.rem(c + 1, NB))

        for tab_sp, vals_v, _ in streams:
            if tab_sp is not None:
                for j in range(CH):
                    pltpu.make_async_copy(tab_sp.at[sidx_v.at[b, j]],
                                          vals_v.at[b, j], semg).wait()
        for tab_sp, vals_v, acc_sp in streams:
            for j in range(CH):
                pltpu.async_copy(val_ref(tab_sp, vals_v, b, j),
                                 acc_sp.at[didx_v.at[b, j]], sems, add=True)

    for cc in range(NCH - NB + 1, NCH):
        drain_scatters(cc % NB)


_IDX2 = pltpu.VMEM((NB, CH, 128), jnp.int32)
_VAL2 = pltpu.VMEM((NB, CH, 128), jnp.float32)
_SEMS = [pltpu.SemaphoreType.DMA] * 3


def _sc_deg_body(dst_hbm, dum_hbm, zeros_hbm, out_hbm,
                 didx_v, ones_v, deg_sp, semi, semg, sems):
    cid = lax.axis_index("c")
    sid = lax.axis_index("s")
    wid = cid * NS + sid
    sl = pl.ds(sid * TSL, TSL)

    for c0 in range(0, 128, 16):
        ones_v[0, pl.ds(c0, 16)] = jnp.full((16,), 1.0, jnp.float32)
    pltpu.sync_copy(zeros_hbm.at[sl], deg_sp.at[sl])
    plsc.subcore_barrier()

    _edge_pipeline(wid, None, dst_hbm, dum_hbm, None, didx_v,
                   [(None, ones_v, deg_sp)], semi, semg, sems)

    plsc.subcore_barrier()
    pltpu.sync_copy(deg_sp.at[sl], out_hbm.at[cid, sl])


def _sc_deg(dstp, dum, zeros_t):
    return pl.kernel(
        _sc_deg_body,
        out_type=jax.ShapeDtypeStruct((NC, T), jnp.float32),
        mesh=_mesh(),
        scratch_types=[_IDX2, pltpu.VMEM((1, 128), jnp.float32),
                       pltpu.VMEM_SHARED((T,), jnp.float32)] + _SEMS,
    )(dstp, dum, zeros_t)


def _sc_agg1_body(src_hbm, dst_hbm, dum_hbm, s_hbm, zeros_hbm, out_hbm,
                  sidx_v, didx_v, vals_v, s_sp, acc_sp, semi, semg, sems):
    cid = lax.axis_index("c")
    sid = lax.axis_index("s")
    wid = cid * NS + sid
    sl = pl.ds(sid * TSL, TSL)

    pltpu.async_copy(s_hbm.at[sl], s_sp.at[sl], semi)
    pltpu.async_copy(zeros_hbm.at[sl], acc_sp.at[sl], semi)
    pltpu.make_async_copy(s_hbm.at[sl], s_sp.at[sl], semi).wait()
    pltpu.make_async_copy(zeros_hbm.at[sl], acc_sp.at[sl], semi).wait()
    plsc.subcore_barrier()

    _edge_pipeline(wid, src_hbm, dst_hbm, dum_hbm, sidx_v, didx_v,
                   [(s_sp, vals_v, acc_sp)], semi, semg, sems)

    plsc.subcore_barrier()
    pltpu.sync_copy(acc_sp.at[sl], out_hbm.at[cid, sl])


def _sc_agg1(srcp, dstp, dum, s_t, zeros_t):
    return pl.kernel(
        _sc_agg1_body,
        out_type=jax.ShapeDtypeStruct((NC, T), jnp.float32),
        mesh=_mesh(),
        scratch_types=[_IDX2, _IDX2, _VAL2,
                       pltpu.VMEM_SHARED((T,), jnp.float32),
                       pltpu.VMEM_SHARED((T,), jnp.float32)] + _SEMS,
    )(srcp, dstp, dum, s_t, zeros_t)


def _sc_agg2_body(src_hbm, dst_hbm, dum_hbm, p0_hbm, p1_hbm, zeros_hbm,
                  out_hbm, sidx_v, didx_v, v0_v, v1_v, p0_sp, p1_sp, a0_sp,
                  a1_sp, semi, semg, sems):
    cid = lax.axis_index("c")
    sid = lax.axis_index("s")
    wid = cid * NS + sid
    sl = pl.ds(sid * TSL, TSL)

    pltpu.async_copy(p0_hbm.at[sl], p0_sp.at[sl], semi)
    pltpu.async_copy(p1_hbm.at[sl], p1_sp.at[sl], semi)
    pltpu.async_copy(zeros_hbm.at[sl], a0_sp.at[sl], semi)
    pltpu.async_copy(zeros_hbm.at[sl], a1_sp.at[sl], semi)
    pltpu.make_async_copy(p0_hbm.at[sl], p0_sp.at[sl], semi).wait()
    pltpu.make_async_copy(p1_hbm.at[sl], p1_sp.at[sl], semi).wait()
    pltpu.make_async_copy(zeros_hbm.at[sl], a0_sp.at[sl], semi).wait()
    pltpu.make_async_copy(zeros_hbm.at[sl], a1_sp.at[sl], semi).wait()
    plsc.subcore_barrier()

    _edge_pipeline(wid, src_hbm, dst_hbm, dum_hbm, sidx_v, didx_v,
                   [(p0_sp, v0_v, a0_sp), (p1_sp, v1_v, a1_sp)],
                   semi, semg, sems)

    plsc.subcore_barrier()
    pltpu.async_copy(a0_sp.at[sl], out_hbm.at[cid, 0, sl], semi)
    pltpu.async_copy(a1_sp.at[sl], out_hbm.at[cid, 1, sl], semi)
    pltpu.make_async_copy(a0_sp.at[sl], out_hbm.at[cid, 0, sl], semi).wait()
    pltpu.make_async_copy(a1_sp.at[sl], out_hbm.at[cid, 1, sl], semi).wait()


def _sc_agg2(srcp, dstp, dum, p0_t, p1_t, zeros_t):
    return pl.kernel(
        _sc_agg2_body,
        out_type=jax.ShapeDtypeStruct((NC, 2, T), jnp.float32),
        mesh=_mesh(),
        scratch_types=[_IDX2, _IDX2, _VAL2, _VAL2]
        + [pltpu.VMEM_SHARED((T,), jnp.float32)] * 4 + _SEMS,
    )(srcp, dstp, dum, p0_t, p1_t, zeros_t)


def _glue1_body(degp_ref, x_ref, dis_ref, s_ref):
    deg = degp_ref[0] + degp_ref[1] + 1.0
    dis = lax.rsqrt(deg)
    dis_ref[...] = dis
    s_ref[...] = dis * x_ref[...]


def _glue1(degp, xpad):
    return pl.pallas_call(
        _glue1_body,
        out_shape=[jax.ShapeDtypeStruct((GR, 128), jnp.float32)] * 2,
    )(degp.reshape(NC, GR, 128), xpad)


def _glue2_body(accp_ref, s_ref, dis_ref, prm_ref, p0_ref, p1_ref):
    dis = dis_ref[...]
    u = dis * (accp_ref[0] + accp_ref[1] + s_ref[...])
    h0 = u * prm_ref[0] + prm_ref[2]
    h1 = u * prm_ref[1] + prm_ref[3]
    p0_ref[...] = dis * jnp.maximum(h0, 0.0)
    p1_ref[...] = dis * jnp.maximum(h1, 0.0)


def _glue2(accp, s, dis, prm1):
    return pl.pallas_call(
        _glue2_body,
        in_specs=[
            pl.BlockSpec(memory_space=pltpu.MemorySpace.VMEM),
            pl.BlockSpec(memory_space=pltpu.MemorySpace.VMEM),
            pl.BlockSpec(memory_space=pltpu.MemorySpace.VMEM),
            pl.BlockSpec(memory_space=pltpu.MemorySpace.SMEM),
        ],
        out_shape=[jax.ShapeDtypeStruct((GR, 128), jnp.float32)] * 2,
    )(accp.reshape(NC, GR, 128), s, dis, prm1)


def _glue3_body(a0_ref, a1_ref, p0_ref, p1_ref, dis_ref, prm_ref,
                o0_ref, o1_ref):
    dis = dis_ref[...]
    t0 = dis * (a0_ref[0] + a0_ref[1] + p0_ref[...])
    t1 = dis * (a1_ref[0] + a1_ref[1] + p1_ref[...])
    o0 = t0 * prm_ref[0] + t1 * prm_ref[2] + prm_ref[4]
    o1 = t0 * prm_ref[1] + t1 * prm_ref[3] + prm_ref[5]
    m = jnp.maximum(o0, o1)
    lse = m + jnp.log(jnp.exp(o0 - m) + jnp.exp(o1 - m))
    o0_ref[...] = o0 - lse
    o1_ref[...] = o1 - lse


def _glue3(a0, a1, p0, p1, dis, prm2):
    return pl.pallas_call(
        _glue3_body,
        in_specs=[
            pl.BlockSpec(memory_space=pltpu.MemorySpace.VMEM),
            pl.BlockSpec(memory_space=pltpu.MemorySpace.VMEM),
            pl.BlockSpec(memory_space=pltpu.MemorySpace.VMEM),
            pl.BlockSpec(memory_space=pltpu.MemorySpace.VMEM),
            pl.BlockSpec(memory_space=pltpu.MemorySpace.VMEM),
            pl.BlockSpec(memory_space=pltpu.MemorySpace.SMEM),
        ],
        out_shape=[jax.ShapeDtypeStruct((GR, 128), jnp.float32)] * 2,
    )(a0, a1, p0, p1, dis, prm2)


@jax.jit
def kernel(x, edge_index, W1, b1, W2, b2):
    srcp = edge_index[0].astype(jnp.int32).reshape(RR, 128)
    dstp = edge_index[1].astype(jnp.int32).reshape(RR, 128)
    dum = jnp.full((CH, 128), N, jnp.int32)

    zeros_t = jnp.zeros((T,), jnp.float32)
    xpad = jnp.pad(x[:, 0], (0, T - N)).reshape(GR, 128)

    degp = _sc_deg(dstp, dum, zeros_t)
    dis, s = _glue1(degp, xpad)

    acc1p = _sc_agg1(srcp, dstp, dum, s.reshape(T), zeros_t)

    prm1 = jnp.concatenate([W1[0], b1]).astype(jnp.float32)
    p0, p1 = _glue2(acc1p, s, dis, prm1)

    acc2p = _sc_agg2(srcp, dstp, dum, p0.reshape(T), p1.reshape(T), zeros_t)

    a0 = acc2p[:, 0, :].reshape(NC, GR, 128)
    a1 = acc2p[:, 1, :].reshape(NC, GR, 128)
    prm2 = jnp.concatenate([W2[0], W2[1], b2]).astype(jnp.float32)
    o0, o1 = _glue3(a0, a1, p0, p1, dis, prm2)

    out = jnp.stack([o0.reshape(T)[:N], o1.reshape(T)[:N]], axis=-1)
    return out
